# Initial kernel scaffold; baseline (speedup 1.0000x reference)
import jax
import jax.numpy as jnp
from jax.experimental import pallas as pl


def kernel(x, table, fc_w, fc_b):
    raise NotImplementedError("write your pallas kernel here")


# speedup vs baseline: 3.2235x; 3.2235x over previous
import functools

import jax
import jax.numpy as jnp
from jax import lax
from jax.experimental import pallas as pl
from jax.experimental.pallas import tpu as pltpu
from jax.experimental.pallas import tpu_sc as plsc

_N_ATTR = 26
_N_VALUES = 1000
_EMBED_DIM = 128
_N_HIDDEN = 1024
_BATCH = 4096

_NC = 2
_NS = 16
_NW = _NC * _NS

_ROWS = _BATCH * _N_ATTR
_RPW = _ROWS // _NW
_CH = 128
_NCH = _RPW // _CH

@functools.cache
def _build_gather_sc():
    mesh = plsc.VectorSubcoreMesh(
        core_axis_name="c", subcore_axis_name="s", num_cores=_NC, num_subcores=_NS
    )

    @functools.partial(
        pl.kernel,
        out_type=jax.ShapeDtypeStruct((_ROWS, _EMBED_DIM), jnp.float32),
        mesh=mesh,
        scratch_types=[
            pltpu.VMEM((_NCH, _CH), jnp.int32),
            pltpu.VMEM((2, _CH, _EMBED_DIM), jnp.float32),
            pltpu.SemaphoreType.DMA,
            pltpu.SemaphoreType.DMA,
            pltpu.SemaphoreType.DMA,
            pltpu.SemaphoreType.DMA,
        ],
    )
    def _gather_sc(idx_hbm, table_hbm, out_hbm, idx_v, rows_v, g0, g1, w0, w1):
        wid = lax.axis_index("s") * _NC + lax.axis_index("c")
        pltpu.sync_copy(idx_hbm.at[wid], idx_v)
        gsems = (g0, g1)
        wsems = (w0, w1)
        out_base = wid * _RPW
        gathers = [None] * _NCH
        writes = [None] * _NCH
        gathers[0] = pltpu.async_copy(table_hbm.at[idx_v.at[0]], rows_v.at[0], gsems[0])
        for j in range(_NCH):
            b = j & 1
            gathers[j].wait()
            if j >= 1:
                writes[j - 1].wait()
            if j + 1 < _NCH:
                gathers[j + 1] = pltpu.async_copy(
                    table_hbm.at[idx_v.at[j + 1]], rows_v.at[b ^ 1], gsems[b ^ 1]
                )
            writes[j] = pltpu.async_copy(
                rows_v.at[b], out_hbm.at[pl.ds(out_base + j * _CH, _CH)], wsems[b]
            )
        writes[_NCH - 1].wait()

    return _gather_sc


_BM = 512


def _mm_body(a_ref, w_ref, b_ref, o_ref):
    o_ref[...] = (
        jnp.dot(a_ref[...], w_ref[...], preferred_element_type=jnp.float32)
        + b_ref[...]
    )


def _matmul_tc(flat, fc_w, fc_b):
    k = _N_ATTR * _EMBED_DIM
    return pl.pallas_call(
        _mm_body,
        grid=(_BATCH // _BM,),
        in_specs=[
            pl.BlockSpec((_BM, k), lambda i: (i, 0)),
            pl.BlockSpec((k, _N_HIDDEN), lambda i: (0, 0)),
            pl.BlockSpec((1, _N_HIDDEN), lambda i: (0, 0)),
        ],
        out_specs=pl.BlockSpec((_BM, _N_HIDDEN), lambda i: (i, 0)),
        out_shape=jax.ShapeDtypeStruct((_BATCH, _N_HIDDEN), jnp.float32),
    )(flat, fc_w, fc_b.reshape(1, _N_HIDDEN))


def kernel(x, table, fc_w, fc_b):
    offs = (jnp.arange(_N_ATTR, dtype=jnp.int32) * _N_VALUES)[None, :]
    idx = (x.astype(jnp.int32) + offs).reshape(_NW, _NCH, _CH)
    emb = _build_gather_sc()(idx, table)
    flat = emb.reshape(_BATCH, _N_ATTR * _EMBED_DIM)
    return _matmul_tc(flat, fc_w, fc_b)
